# baseline (speedup 1.0000x reference)
import jax
import jax.numpy as jnp
from jax import lax
from jax.experimental import pallas as pl
from jax.experimental.pallas import tpu as pltpu
from jax.experimental.pallas import tpu_sc as plsc

N = 160000
D = 256
H = 256
S = 10000


_BN = 2048


def _gate_body(a_ref, w1_ref, b1_ref, w2_ref, b2_ref, o_ref):
    a = a_ref[...]
    h = jnp.dot(a, w1_ref[...], preferred_element_type=jnp.float32)
    h = jnp.maximum(h + b1_ref[...], 0.0)
    z = jnp.sum(h * w2_ref[...], axis=1) + b2_ref[...]
    o_ref[...] = 1.0 / (1.0 + jnp.exp(-z))


def _gates(atom_feats, W1, b1, W2, b2):
    grid = -(-N // _BN)
    return pl.pallas_call(
        _gate_body,
        grid=(grid,),
        in_specs=[
            pl.BlockSpec((_BN, D), lambda i: (i, 0)),
            pl.BlockSpec((D, H), lambda i: (0, 0)),
            pl.BlockSpec((H,), lambda i: (0,)),
            pl.BlockSpec((1, H), lambda i: (0, 0)),
            pl.BlockSpec((1,), lambda i: (0,)),
        ],
        out_specs=pl.BlockSpec((_BN,), lambda i: (i,)),
        out_shape=jax.ShapeDtypeStruct((N,), jnp.float32),
    )(atom_feats, W1, b1, W2.reshape(1, H), b2)


_NC = 2
_NS = 16
_SP = 10240


_CHA = 640
_NCHA = N // _CHA
_ITA = -(-_NCHA // _NS)


def _gsum_body(g_hbm, b_hbm, out_hbm, gacc, grep, g_v, idx3, sem_g, sem_s):
    s = lax.axis_index("s")
    c = lax.axis_index("c")
    del c

    def _zrow(r, carry):
        grep[0, r, pl.ds(0, 16)] = jnp.zeros((16,), jnp.float32)
        return carry
    lax.fori_loop(0, _CHA, _zrow, 0)
    pltpu.sync_copy(grep.at[0], gacc.at[pl.ds(s * _CHA, _CHA)])
    plsc.subcore_barrier()

    def _issue_gather(j, b2, b4):
        row0 = (j * _NS + s) * _CHA
        pltpu.async_copy(g_hbm.at[pl.ds(row0, _CHA)], g_v.at[b2],
                         sem_g.at[b2])
        for k in range(5):
            pltpu.async_copy(b_hbm.at[pl.ds(row0 + k * 128, 128)],
                             idx3.at[b4, k], sem_g.at[b2])

    def _wait_gather(j, b2, b4):
        row0 = (j * _NS + s) * _CHA
        pltpu.make_async_copy(g_hbm.at[pl.ds(row0, _CHA)], g_v.at[b2],
                              sem_g.at[b2]).wait()
        for k in range(5):
            pltpu.make_async_copy(b_hbm.at[pl.ds(row0 + k * 128, 128)],
                                  idx3.at[b4, k], sem_g.at[b2]).wait()

    def _issue_scatter(b2, b4):
        for k in range(5):
            pltpu.async_copy(grep.at[b2, pl.ds(k * 128, 128)],
                             gacc.at[idx3.at[b4, k]], sem_s.at[b2], add=True)

    def _wait_scatter(b2, b4):
        for k in range(5):
            pltpu.make_async_copy(grep.at[b2, pl.ds(k * 128, 128)],
                                  gacc.at[idx3.at[b4, k]],
                                  sem_s.at[b2]).wait()

    _issue_gather(0, 0, 0)
    _issue_gather(1, 1, 1)

    def _group(g, carry):
        for b in range(4):
            j = g * 4 + b
            b2 = b % 2

            @pl.when(j * _NS + s < _NCHA)
            def _():
                _wait_gather(j, b2, b)

                @pl.when(j >= 2)
                def _():
                    _wait_scatter(b2, (b + 2) % 4)

                def _row16(r16, carry2):
                    gvec = g_v[b2, pl.ds(r16 * 16, 16)]
                    for k in range(16):
                        grep[b2, r16 * 16 + k, pl.ds(0, 16)] = jnp.full(
                            (16,), gvec[k], jnp.float32)
                    return carry2
                lax.fori_loop(0, _CHA // 16, _row16, 0)

                _issue_scatter(b2, b)

                @pl.when((j + 2) * _NS + s < _NCHA)
                def _():
                    _issue_gather(j + 2, b2, (b + 2) % 4)
        return carry
    lax.fori_loop(0, -(-_ITA // 4), _group, 0)

    _wait_scatter(0, 0)
    _wait_scatter(1, 1)
    plsc.subcore_barrier()

    pltpu.sync_copy(gacc.at[pl.ds(s * _CHA, _CHA)], grep.at[0])
    pltpu.sync_copy(grep.at[0], out_hbm.at[pl.ds(s * _CHA, _CHA), :])


def _gate_sums(gates, batch_i32):
    mesh = plsc.VectorSubcoreMesh(core_axis_name="c", subcore_axis_name="s")
    f = pl.kernel(
        _gsum_body,
        out_type=jax.ShapeDtypeStruct((_SP, 16), jnp.float32),
        mesh=mesh,
        scratch_types=[
            pltpu.VMEM_SHARED((_SP, 16), jnp.float32),
            pltpu.VMEM((2, _CHA, 16), jnp.float32),
            pltpu.VMEM((2, _CHA), jnp.float32),
            pltpu.VMEM((4, 5, 128), jnp.int32),
            pltpu.SemaphoreType.DMA((2,)),
            pltpu.SemaphoreType.DMA((2,)),
        ],
        compiler_params=pltpu.CompilerParams(use_tc_tiling_on_sc=False),
    )
    return f(gates, batch_i32)


_CH = 80
_NCHUNK = N // _CH
_ITERS = -(-_NCHUNK // _NS)
_SROWS = _SP // _NS
_NB = 4


def _sc_body(a_hbm, g_hbm, b_hbm, gs_hbm, out_hbm, acc, buf4, g_v4, idx_v4,
             gs_v, sem_g, sem_s):
    c = lax.axis_index("c")
    s = lax.axis_index("s")
    col0 = c * 128

    def _zrow(r, carry):
        for j in range(8):
            buf4[0, r, pl.ds(j * 16, 16)] = jnp.zeros((16,), jnp.float32)
        return carry
    lax.fori_loop(0, _CH, _zrow, 0)
    zbase = s * _SROWS
    for i in range(_SROWS // _CH):
        pltpu.sync_copy(buf4.at[0], acc.at[pl.ds(zbase + i * _CH, _CH)])
    plsc.subcore_barrier()

    def _issue_gather(j, b):
        row0 = (j * _NS + s) * _CH
        pltpu.async_copy(a_hbm.at[pl.ds(row0, _CH), pl.ds(col0, 128)],
                         buf4.at[b], sem_g.at[b])
        pltpu.async_copy(g_hbm.at[pl.ds(row0, _CH)], g_v4.at[b], sem_g.at[b])
        pltpu.async_copy(b_hbm.at[pl.ds(row0, _CH)], idx_v4.at[b], sem_g.at[b])

    def _wait_gather(j, b):
        row0 = (j * _NS + s) * _CH
        pltpu.make_async_copy(a_hbm.at[pl.ds(row0, _CH), pl.ds(col0, 128)],
                              buf4.at[b], sem_g.at[b]).wait()
        pltpu.make_async_copy(g_hbm.at[pl.ds(row0, _CH)], g_v4.at[b],
                              sem_g.at[b]).wait()
        pltpu.make_async_copy(b_hbm.at[pl.ds(row0, _CH)], idx_v4.at[b],
                              sem_g.at[b]).wait()

    def _issue_scatter(b):
        pltpu.async_copy(buf4.at[b], acc.at[idx_v4.at[b]], sem_s.at[b],
                         add=True)

    def _wait_scatter(b):
        pltpu.make_async_copy(buf4.at[b], acc.at[idx_v4.at[b]],
                              sem_s.at[b]).wait()

    _issue_gather(0, 0)
    _issue_gather(1, 1)

    def _group(g, carry):
        for b in range(_NB):
            j = g * _NB + b

            @pl.when(j * _NS + s < _NCHUNK)
            def _():
                _wait_gather(j, b)

                @plsc.parallel_loop(0, _CH // 16, unroll=2)
                def _row16(r16):
                    gvec = g_v4[b, pl.ds(r16 * 16, 16)]
                    for k in range(16):
                        gk = gvec[k]
                        r = r16 * 16 + k
                        for jj in range(8):
                            buf4[b, r, pl.ds(jj * 16, 16)] = (
                                buf4[b, r, pl.ds(jj * 16, 16)] * gk)

                _issue_scatter(b)

                b2 = (b + 2) % _NB

                @pl.when((j + 2) * _NS + s < _NCHUNK)
                def _():
                    @pl.when(j >= 2)
                    def _():
                        _wait_scatter(b2)
                    _issue_gather(j + 2, b2)
        return carry
    lax.fori_loop(0, -(-_ITERS // _NB), _group, 0)

    for b in range(_NB):
        _wait_scatter(b)
    plsc.subcore_barrier()

    base = s * _SROWS
    for sub in range(_SROWS // _CH):
        r0 = base + sub * _CH
        pltpu.sync_copy(acc.at[pl.ds(r0, _CH)], buf4.at[0])
        pltpu.sync_copy(gs_hbm.at[pl.ds(r0 * 16, _CH * 16)], gs_v)

        def _div(r, carry2):
            gv = gs_v[pl.ds(r * 16, 16)]
            recip = jnp.ones((16,), jnp.float32) / (gv + 1e-6)
            for j in range(8):
                buf4[0, r, pl.ds(j * 16, 16)] = (
                    buf4[0, r, pl.ds(j * 16, 16)] * recip)
            return carry2
        lax.fori_loop(0, _CH, _div, 0)

        @pl.when(r0 + _CH <= S)
        def _():
            pltpu.sync_copy(buf4.at[0],
                            out_hbm.at[pl.ds(r0, _CH), pl.ds(col0, 128)])

        if S % _CH:
            @pl.when(jnp.logical_and(r0 < S, r0 + _CH > S))
            def _():
                pltpu.sync_copy(
                    buf4.at[0, pl.ds(0, S % _CH)],
                    out_hbm.at[pl.ds(r0, S % _CH), pl.ds(col0, 128)])


def _pool(atom_feats, gates, batch_i32, gsums):
    mesh = plsc.VectorSubcoreMesh(core_axis_name="c", subcore_axis_name="s")
    f = pl.kernel(
        _sc_body,
        out_type=jax.ShapeDtypeStruct((S, D), jnp.float32),
        mesh=mesh,
        scratch_types=[
            pltpu.VMEM_SHARED((_SP, 128), jnp.float32),
            pltpu.VMEM((_NB, _CH, 128), jnp.float32),
            pltpu.VMEM((_NB, _CH), jnp.float32),
            pltpu.VMEM((_NB, _CH), jnp.int32),
            pltpu.VMEM((_CH * 16,), jnp.float32),
            pltpu.SemaphoreType.DMA((_NB,)),
            pltpu.SemaphoreType.DMA((_NB,)),
        ],
    )
    return f(atom_feats, gates, batch_i32, gsums.reshape(-1))


def kernel(atom_feats, batch, W1, b1, W2, b2):
    gates = _gates(atom_feats, W1, b1, W2, b2)
    batch_i32 = batch.astype(jnp.int32)
    gsums = _gate_sums(gates, batch_i32)
    return _pool(atom_feats, gates, batch_i32, gsums)


# speedup vs baseline: 1.0283x; 1.0283x over previous
import jax
import jax.numpy as jnp
from jax import lax
from jax.experimental import pallas as pl
from jax.experimental.pallas import tpu as pltpu
from jax.experimental.pallas import tpu_sc as plsc

N = 160000
D = 256
H = 256
S = 10000


_BN = 2048


def _gate_body(a_ref, w1_ref, b1_ref, w2_ref, b2_ref, o_ref):
    a = a_ref[...]
    h = jnp.dot(a, w1_ref[...], preferred_element_type=jnp.float32)
    h = jnp.maximum(h + b1_ref[...], 0.0)
    z = jnp.sum(h * w2_ref[...], axis=1) + b2_ref[...]
    o_ref[...] = 1.0 / (1.0 + jnp.exp(-z))


def _gates(atom_feats, W1, b1, W2, b2):
    grid = -(-N // _BN)
    return pl.pallas_call(
        _gate_body,
        grid=(grid,),
        in_specs=[
            pl.BlockSpec((_BN, D), lambda i: (i, 0)),
            pl.BlockSpec((D, H), lambda i: (0, 0)),
            pl.BlockSpec((H,), lambda i: (0,)),
            pl.BlockSpec((1, H), lambda i: (0, 0)),
            pl.BlockSpec((1,), lambda i: (0,)),
        ],
        out_specs=pl.BlockSpec((_BN,), lambda i: (i,)),
        out_shape=jax.ShapeDtypeStruct((N,), jnp.float32),
    )(atom_feats, W1, b1, W2.reshape(1, H), b2)


_NC = 2
_NS = 16
_SP = 10240


_CHA = 640
_NCHA = N // _CHA
_ITA = -(-_NCHA // _NS)


def _gsum_body(g_hbm, b_hbm, out_hbm, gacc, grep, g_v, idx3, sem_g, sem_s):
    s = lax.axis_index("s")
    c = lax.axis_index("c")
    del c

    def _zrow(r, carry):
        grep[0, r, pl.ds(0, 16)] = jnp.zeros((16,), jnp.float32)
        return carry
    lax.fori_loop(0, _CHA, _zrow, 0)
    pltpu.sync_copy(grep.at[0], gacc.at[pl.ds(s * _CHA, _CHA)])
    plsc.subcore_barrier()

    def _issue_gather(j, b2, b4):
        row0 = (j * _NS + s) * _CHA
        pltpu.async_copy(g_hbm.at[pl.ds(row0, _CHA)], g_v.at[b2],
                         sem_g.at[b2])
        for k in range(5):
            pltpu.async_copy(b_hbm.at[pl.ds(row0 + k * 128, 128)],
                             idx3.at[b4, k], sem_g.at[b2])

    def _wait_gather(j, b2, b4):
        row0 = (j * _NS + s) * _CHA
        pltpu.make_async_copy(g_hbm.at[pl.ds(row0, _CHA)], g_v.at[b2],
                              sem_g.at[b2]).wait()
        for k in range(5):
            pltpu.make_async_copy(b_hbm.at[pl.ds(row0 + k * 128, 128)],
                                  idx3.at[b4, k], sem_g.at[b2]).wait()

    def _issue_scatter(b2, b4):
        for k in range(5):
            pltpu.async_copy(grep.at[b2, pl.ds(k * 128, 128)],
                             gacc.at[idx3.at[b4, k]], sem_s.at[b2], add=True)

    def _wait_scatter(b2, b4):
        for k in range(5):
            pltpu.make_async_copy(grep.at[b2, pl.ds(k * 128, 128)],
                                  gacc.at[idx3.at[b4, k]],
                                  sem_s.at[b2]).wait()

    _issue_gather(0, 0, 0)
    _issue_gather(1, 1, 1)

    def _group(g, carry):
        for b in range(4):
            j = g * 4 + b
            b2 = b % 2

            @pl.when(j * _NS + s < _NCHA)
            def _():
                _wait_gather(j, b2, b)

                @pl.when(j >= 2)
                def _():
                    _wait_scatter(b2, (b + 2) % 4)

                def _row16(r16, carry2):
                    gvec = g_v[b2, pl.ds(r16 * 16, 16)]
                    for k in range(16):
                        grep[b2, r16 * 16 + k, pl.ds(0, 16)] = jnp.full(
                            (16,), gvec[k], jnp.float32)
                    return carry2
                lax.fori_loop(0, _CHA // 16, _row16, 0)

                _issue_scatter(b2, b)

                @pl.when((j + 2) * _NS + s < _NCHA)
                def _():
                    _issue_gather(j + 2, b2, (b + 2) % 4)
        return carry
    lax.fori_loop(0, -(-_ITA // 4), _group, 0)

    _wait_scatter(0, 0)
    _wait_scatter(1, 1)
    plsc.subcore_barrier()

    pltpu.sync_copy(gacc.at[pl.ds(s * _CHA, _CHA)], grep.at[0])
    pltpu.sync_copy(grep.at[0], out_hbm.at[pl.ds(s * _CHA, _CHA), :])


def _gate_sums(gates, batch_i32):
    mesh = plsc.VectorSubcoreMesh(core_axis_name="c", subcore_axis_name="s")
    f = pl.kernel(
        _gsum_body,
        out_type=jax.ShapeDtypeStruct((_SP, 16), jnp.float32),
        mesh=mesh,
        scratch_types=[
            pltpu.VMEM_SHARED((_SP, 16), jnp.float32),
            pltpu.VMEM((2, _CHA, 16), jnp.float32),
            pltpu.VMEM((2, _CHA), jnp.float32),
            pltpu.VMEM((4, 5, 128), jnp.int32),
            pltpu.SemaphoreType.DMA((2,)),
            pltpu.SemaphoreType.DMA((2,)),
        ],
        compiler_params=pltpu.CompilerParams(use_tc_tiling_on_sc=False),
    )
    return f(gates, batch_i32)


_CH = 64
_NCHUNK = N // _CH
_ITERS = -(-_NCHUNK // _NS)
_SROWS = _SP // _NS
_NB = 4


def _sc_body(a_hbm, g_hbm, b_hbm, gs_hbm, out_hbm, acc, buf4, g_v4, idx_v4,
             gs_v, sem_g, sem_s):
    c = lax.axis_index("c")
    s = lax.axis_index("s")
    col0 = c * 128

    def _zrow(r, carry):
        for j in range(8):
            buf4[0, r, pl.ds(j * 16, 16)] = jnp.zeros((16,), jnp.float32)
        return carry
    lax.fori_loop(0, _CH, _zrow, 0)
    zbase = s * _SROWS
    for i in range(_SROWS // _CH):
        pltpu.sync_copy(buf4.at[0], acc.at[pl.ds(zbase + i * _CH, _CH)])
    plsc.subcore_barrier()

    def _issue_gather(j, b):
        row0 = (j * _NS + s) * _CH
        pltpu.async_copy(a_hbm.at[pl.ds(row0, _CH), pl.ds(col0, 128)],
                         buf4.at[b], sem_g.at[b])
        pltpu.async_copy(g_hbm.at[pl.ds(row0, _CH)], g_v4.at[b], sem_g.at[b])
        pltpu.async_copy(b_hbm.at[pl.ds(row0, _CH)], idx_v4.at[b], sem_g.at[b])

    def _wait_gather(j, b):
        row0 = (j * _NS + s) * _CH
        pltpu.make_async_copy(a_hbm.at[pl.ds(row0, _CH), pl.ds(col0, 128)],
                              buf4.at[b], sem_g.at[b]).wait()
        pltpu.make_async_copy(g_hbm.at[pl.ds(row0, _CH)], g_v4.at[b],
                              sem_g.at[b]).wait()
        pltpu.make_async_copy(b_hbm.at[pl.ds(row0, _CH)], idx_v4.at[b],
                              sem_g.at[b]).wait()

    def _issue_scatter(b):
        pltpu.async_copy(buf4.at[b], acc.at[idx_v4.at[b]], sem_s.at[b],
                         add=True)

    def _wait_scatter(b):
        pltpu.make_async_copy(buf4.at[b], acc.at[idx_v4.at[b]],
                              sem_s.at[b]).wait()

    _issue_gather(0, 0)
    _issue_gather(1, 1)

    def _group(g, carry):
        for b in range(_NB):
            j = g * _NB + b

            @pl.when(j * _NS + s < _NCHUNK)
            def _():
                _wait_gather(j, b)

                @plsc.parallel_loop(0, _CH // 16, unroll=2)
                def _row16(r16):
                    gvec = g_v4[b, pl.ds(r16 * 16, 16)]
                    for k in range(16):
                        gk = gvec[k]
                        r = r16 * 16 + k
                        for jj in range(8):
                            buf4[b, r, pl.ds(jj * 16, 16)] = (
                                buf4[b, r, pl.ds(jj * 16, 16)] * gk)

                _issue_scatter(b)

                b2 = (b + 2) % _NB

                @pl.when((j + 2) * _NS + s < _NCHUNK)
                def _():
                    @pl.when(j >= 2)
                    def _():
                        _wait_scatter(b2)
                    _issue_gather(j + 2, b2)
        return carry
    lax.fori_loop(0, -(-_ITERS // _NB), _group, 0)

    for b in range(_NB):
        _wait_scatter(b)
    plsc.subcore_barrier()

    base = s * _SROWS
    for sub in range(_SROWS // _CH):
        r0 = base + sub * _CH
        pltpu.sync_copy(acc.at[pl.ds(r0, _CH)], buf4.at[0])
        pltpu.sync_copy(gs_hbm.at[pl.ds(r0 * 16, _CH * 16)], gs_v)

        def _div(r, carry2):
            gv = gs_v[pl.ds(r * 16, 16)]
            recip = jnp.ones((16,), jnp.float32) / (gv + 1e-6)
            for j in range(8):
                buf4[0, r, pl.ds(j * 16, 16)] = (
                    buf4[0, r, pl.ds(j * 16, 16)] * recip)
            return carry2
        lax.fori_loop(0, _CH, _div, 0)

        @pl.when(r0 + _CH <= S)
        def _():
            pltpu.sync_copy(buf4.at[0],
                            out_hbm.at[pl.ds(r0, _CH), pl.ds(col0, 128)])

        if S % _CH:
            @pl.when(jnp.logical_and(r0 < S, r0 + _CH > S))
            def _():
                pltpu.sync_copy(
                    buf4.at[0, pl.ds(0, S % _CH)],
                    out_hbm.at[pl.ds(r0, S % _CH), pl.ds(col0, 128)])


def _pool(atom_feats, gates, batch_i32, gsums):
    mesh = plsc.VectorSubcoreMesh(core_axis_name="c", subcore_axis_name="s")
    f = pl.kernel(
        _sc_body,
        out_type=jax.ShapeDtypeStruct((S, D), jnp.float32),
        mesh=mesh,
        scratch_types=[
            pltpu.VMEM_SHARED((_SP, 128), jnp.float32),
            pltpu.VMEM((_NB, _CH, 128), jnp.float32),
            pltpu.VMEM((_NB, _CH), jnp.float32),
            pltpu.VMEM((_NB, _CH), jnp.int32),
            pltpu.VMEM((_CH * 16,), jnp.float32),
            pltpu.SemaphoreType.DMA((_NB,)),
            pltpu.SemaphoreType.DMA((_NB,)),
        ],
    )
    return f(atom_feats, gates, batch_i32, gsums.reshape(-1))


def kernel(atom_feats, batch, W1, b1, W2, b2):
    gates = _gates(atom_feats, W1, b1, W2, b2)
    batch_i32 = batch.astype(jnp.int32)
    gsums = _gate_sums(gates, batch_i32)
    return _pool(atom_feats, gates, batch_i32, gsums)


# speedup vs baseline: 1.0406x; 1.0119x over previous
import jax
import jax.numpy as jnp
from jax import lax
from jax.experimental import pallas as pl
from jax.experimental.pallas import tpu as pltpu
from jax.experimental.pallas import tpu_sc as plsc

N = 160000
D = 256
H = 256
S = 10000


_BN = 2048


def _gate_body(a_ref, w1_ref, b1_ref, w2_ref, b2_ref, o_ref):
    a = a_ref[...]
    h = jnp.dot(a, w1_ref[...], preferred_element_type=jnp.float32)
    h = jnp.maximum(h + b1_ref[...], 0.0)
    z = jnp.sum(h * w2_ref[...], axis=1) + b2_ref[...]
    o_ref[...] = 1.0 / (1.0 + jnp.exp(-z))


def _gates(atom_feats, W1, b1, W2, b2):
    grid = -(-N // _BN)
    return pl.pallas_call(
        _gate_body,
        grid=(grid,),
        in_specs=[
            pl.BlockSpec((_BN, D), lambda i: (i, 0)),
            pl.BlockSpec((D, H), lambda i: (0, 0)),
            pl.BlockSpec((H,), lambda i: (0,)),
            pl.BlockSpec((1, H), lambda i: (0, 0)),
            pl.BlockSpec((1,), lambda i: (0,)),
        ],
        out_specs=pl.BlockSpec((_BN,), lambda i: (i,)),
        out_shape=jax.ShapeDtypeStruct((N,), jnp.float32),
    )(atom_feats, W1, b1, W2.reshape(1, H), b2)


_NC = 2
_NS = 16
_SP = 10240


_CHA = 640
_NCHA = N // _CHA
_ITA = -(-_NCHA // _NS)


def _gsum_body(g_hbm, b_hbm, out_hbm, gacc, grep, g_v, idx3, sem_g, sem_s):
    s = lax.axis_index("s")
    c = lax.axis_index("c")
    del c

    def _zrow(r, carry):
        grep[0, r, pl.ds(0, 16)] = jnp.zeros((16,), jnp.float32)
        return carry
    lax.fori_loop(0, _CHA, _zrow, 0)
    pltpu.sync_copy(grep.at[0], gacc.at[pl.ds(s * _CHA, _CHA)])
    plsc.subcore_barrier()

    def _issue_gather(j, b2, b4):
        row0 = (j * _NS + s) * _CHA
        pltpu.async_copy(g_hbm.at[pl.ds(row0, _CHA)], g_v.at[b2],
                         sem_g.at[b2])
        for k in range(5):
            pltpu.async_copy(b_hbm.at[pl.ds(row0 + k * 128, 128)],
                             idx3.at[b4, k], sem_g.at[b2])

    def _wait_gather(j, b2, b4):
        row0 = (j * _NS + s) * _CHA
        pltpu.make_async_copy(g_hbm.at[pl.ds(row0, _CHA)], g_v.at[b2],
                              sem_g.at[b2]).wait()
        for k in range(5):
            pltpu.make_async_copy(b_hbm.at[pl.ds(row0 + k * 128, 128)],
                                  idx3.at[b4, k], sem_g.at[b2]).wait()

    def _issue_scatter(b2, b4):
        for k in range(5):
            pltpu.async_copy(grep.at[b2, pl.ds(k * 128, 128)],
                             gacc.at[idx3.at[b4, k]], sem_s.at[b2], add=True)

    def _wait_scatter(b2, b4):
        for k in range(5):
            pltpu.make_async_copy(grep.at[b2, pl.ds(k * 128, 128)],
                                  gacc.at[idx3.at[b4, k]],
                                  sem_s.at[b2]).wait()

    _issue_gather(0, 0, 0)
    _issue_gather(1, 1, 1)

    def _group(g, carry):
        for b in range(4):
            j = g * 4 + b
            b2 = b % 2

            @pl.when(j * _NS + s < _NCHA)
            def _():
                _wait_gather(j, b2, b)

                @pl.when(j >= 2)
                def _():
                    _wait_scatter(b2, (b + 2) % 4)

                def _row16(r16, carry2):
                    gvec = g_v[b2, pl.ds(r16 * 16, 16)]
                    for k in range(16):
                        grep[b2, r16 * 16 + k, pl.ds(0, 16)] = jnp.full(
                            (16,), gvec[k], jnp.float32)
                    return carry2
                lax.fori_loop(0, _CHA // 16, _row16, 0)

                _issue_scatter(b2, b)

                @pl.when((j + 2) * _NS + s < _NCHA)
                def _():
                    _issue_gather(j + 2, b2, (b + 2) % 4)
        return carry
    lax.fori_loop(0, -(-_ITA // 4), _group, 0)

    _wait_scatter(0, 0)
    _wait_scatter(1, 1)
    plsc.subcore_barrier()

    pltpu.sync_copy(gacc.at[pl.ds(s * _CHA, _CHA)], grep.at[0])
    pltpu.sync_copy(grep.at[0], out_hbm.at[pl.ds(s * _CHA, _CHA), :])


def _gate_sums(gates, batch_i32):
    mesh = plsc.VectorSubcoreMesh(core_axis_name="c", subcore_axis_name="s")
    f = pl.kernel(
        _gsum_body,
        out_type=jax.ShapeDtypeStruct((_SP, 16), jnp.float32),
        mesh=mesh,
        scratch_types=[
            pltpu.VMEM_SHARED((_SP, 16), jnp.float32),
            pltpu.VMEM((2, _CHA, 16), jnp.float32),
            pltpu.VMEM((2, _CHA), jnp.float32),
            pltpu.VMEM((4, 5, 128), jnp.int32),
            pltpu.SemaphoreType.DMA((2,)),
            pltpu.SemaphoreType.DMA((2,)),
        ],
        compiler_params=pltpu.CompilerParams(use_tc_tiling_on_sc=False),
    )
    return f(gates, batch_i32)


_CH = 64
_NCHUNK = N // _CH
_ITERS = -(-_NCHUNK // _NS)
_SROWS = _SP // _NS
_NB = 4


def _sc_body(a_hbm, g_hbm, b_hbm, gs_hbm, out_hbm, acc, buf4, g_v4, idx_v4,
             gs_v, sem_g, sem_s):
    c = lax.axis_index("c")
    s = lax.axis_index("s")
    col0 = c * 128

    def _zrow(r, carry):
        for j in range(8):
            buf4[0, r, pl.ds(j * 16, 16)] = jnp.zeros((16,), jnp.float32)
        return carry
    lax.fori_loop(0, _CH, _zrow, 0)
    zbase = s * _SROWS
    for i in range(_SROWS // _CH):
        pltpu.sync_copy(buf4.at[0], acc.at[pl.ds(zbase + i * _CH, _CH)])
    plsc.subcore_barrier()

    def _issue_gather(j, b):
        row0 = (j * _NS + s) * _CH
        pltpu.async_copy(a_hbm.at[pl.ds(row0, _CH), pl.ds(col0, 128)],
                         buf4.at[b], sem_g.at[b])
        pltpu.async_copy(g_hbm.at[pl.ds(row0, _CH)], g_v4.at[b], sem_g.at[b])
        pltpu.async_copy(b_hbm.at[pl.ds(row0, _CH)], idx_v4.at[b], sem_g.at[b])

    def _wait_gather(j, b):
        row0 = (j * _NS + s) * _CH
        pltpu.make_async_copy(a_hbm.at[pl.ds(row0, _CH), pl.ds(col0, 128)],
                              buf4.at[b], sem_g.at[b]).wait()
        pltpu.make_async_copy(g_hbm.at[pl.ds(row0, _CH)], g_v4.at[b],
                              sem_g.at[b]).wait()
        pltpu.make_async_copy(b_hbm.at[pl.ds(row0, _CH)], idx_v4.at[b],
                              sem_g.at[b]).wait()

    def _issue_scatter(b):
        pltpu.async_copy(buf4.at[b], acc.at[idx_v4.at[b]], sem_s.at[b],
                         add=True)

    def _wait_scatter(b):
        pltpu.make_async_copy(buf4.at[b], acc.at[idx_v4.at[b]],
                              sem_s.at[b]).wait()

    _issue_gather(0, 0)
    _issue_gather(1, 1)

    def _group(g, carry):
        for b in range(_NB):
            j = g * _NB + b

            @pl.when(j * _NS + s < _NCHUNK)
            def _():
                _wait_gather(j, b)

                @plsc.parallel_loop(0, _CH // 16, unroll=2)
                def _row16(r16):
                    gvec = g_v4[b, pl.ds(r16 * 16, 16)]
                    for k in range(16):
                        gk = gvec[k]
                        r = r16 * 16 + k
                        for jj in range(8):
                            buf4[b, r, pl.ds(jj * 16, 16)] = (
                                buf4[b, r, pl.ds(jj * 16, 16)] * gk)

                _issue_scatter(b)

                b2 = (b + 2) % _NB

                @pl.when((j + 2) * _NS + s < _NCHUNK)
                def _():
                    @pl.when(j >= 2)
                    def _():
                        _wait_scatter(b2)
                    _issue_gather(j + 2, b2)
        return carry
    lax.fori_loop(0, -(-_ITERS // _NB), _group, 0)

    for b in range(_NB):
        _wait_scatter(b)
    plsc.subcore_barrier()

    base = s * _SROWS
    for sub in range(_SROWS // _CH):
        r0 = base + sub * _CH
        pltpu.sync_copy(acc.at[pl.ds(r0, _CH)], buf4.at[0])
        pltpu.sync_copy(gs_hbm.at[pl.ds(r0, _CH), :], gs_v)

        def _div(r, carry2):
            gv = gs_v[r, pl.ds(0, 16)]
            recip = jnp.ones((16,), jnp.float32) / (gv + 1e-6)
            for j in range(8):
                buf4[0, r, pl.ds(j * 16, 16)] = (
                    buf4[0, r, pl.ds(j * 16, 16)] * recip)
            return carry2
        lax.fori_loop(0, _CH, _div, 0)

        @pl.when(r0 + _CH <= S)
        def _():
            pltpu.sync_copy(buf4.at[0],
                            out_hbm.at[pl.ds(r0, _CH), pl.ds(col0, 128)])

        if S % _CH:
            @pl.when(jnp.logical_and(r0 < S, r0 + _CH > S))
            def _():
                pltpu.sync_copy(
                    buf4.at[0, pl.ds(0, S % _CH)],
                    out_hbm.at[pl.ds(r0, S % _CH), pl.ds(col0, 128)])


def _pool(atom_feats, gates, batch_i32, gsums):
    mesh = plsc.VectorSubcoreMesh(core_axis_name="c", subcore_axis_name="s")
    f = pl.kernel(
        _sc_body,
        out_type=jax.ShapeDtypeStruct((S, D), jnp.float32),
        mesh=mesh,
        scratch_types=[
            pltpu.VMEM_SHARED((_SP, 128), jnp.float32),
            pltpu.VMEM((_NB, _CH, 128), jnp.float32),
            pltpu.VMEM((_NB, _CH), jnp.float32),
            pltpu.VMEM((_NB, _CH), jnp.int32),
            pltpu.VMEM((_CH, 16), jnp.float32),
            pltpu.SemaphoreType.DMA((_NB,)),
            pltpu.SemaphoreType.DMA((_NB,)),
        ],
    )
    return f(atom_feats, gates, batch_i32, gsums)


def kernel(atom_feats, batch, W1, b1, W2, b2):
    gates = _gates(atom_feats, W1, b1, W2, b2)
    batch_i32 = batch.astype(jnp.int32)
    gsums = _gate_sums(gates, batch_i32)
    return _pool(atom_feats, gates, batch_i32, gsums)
